# baseline (speedup 1.0000x reference)
import functools

import jax
import jax.numpy as jnp
from jax import lax
from jax.experimental import pallas as pl
from jax.experimental.pallas import tpu as pltpu
from jax.experimental.pallas import tpu_sc as plsc

N_NODES = 100000
EMBED_DIM = 32
HALF_DIM = 16
N_EDGES = 1600000

EPB = 128
N_BLOCKS = N_EDGES // EPB
NS = 16
CPT = -(-N_BLOCKS // NS)
GRP = 4
PPG = GRP * EPB
DBLK = 200
N_DB = N_NODES // DBLK
DBPT = -(-N_DB // NS)


def _mul_block(rowsb, wb, roff):
    def grp16(g, c):
        base = roff + g * HALF_DIM
        wv = wb[pl.ds(base, HALF_DIM)]
        for k in range(HALF_DIM):
            splat = jnp.take_along_axis(
                wv, jnp.full((HALF_DIM,), k, jnp.int32), axis=0,
                mode="promise_in_bounds")
            rowsb[base + k, :] = rowsb[base + k, :] * splat
        return c
    lax.fori_loop(0, EPB // HALF_DIM, grp16, 0)


def _layer(cid, tid, cur, out, src, dst, w, last, x1, x2,
           srcb, dstb, wb, rowsb, acc, sem_e, sem_g, sem_s):
    scope_zero = jax.named_scope("phase_zero")
    scope_zero.__enter__()
    def zb(e, c):
        rowsb[e, :] = jnp.zeros((HALF_DIM,), jnp.float32)
        return c
    lax.fori_loop(0, DBLK, zb, 0, unroll=8)
    nzb = jnp.minimum(DBPT, jnp.maximum(0, N_DB - tid * DBPT))

    def zissue(i, c):
        r0 = pl.multiple_of((tid * DBPT + i) * DBLK, 8)
        pltpu.async_copy(rowsb.at[pl.ds(0, DBLK)], acc.at[pl.ds(r0, DBLK)],
                         sem_s)
        return c
    lax.fori_loop(0, nzb, zissue, 0)

    def zdrain(i, c):
        pltpu.make_async_copy(rowsb.at[pl.ds(0, DBLK)],
                              acc.at[pl.ds(0, DBLK)], sem_s).wait()
        return c
    lax.fori_loop(0, nzb, zdrain, 0)
    plsc.subcore_barrier()
    scope_zero.__exit__(None, None, None)
    scope_edge = jax.named_scope("phase_edge")
    scope_edge.__enter__()

    nch = jnp.minimum(CPT, jnp.maximum(0, N_BLOCKS - tid * CPT))
    ngrp = nch // GRP
    npair = ngrp // 2

    def off_of(g):
        blk = jnp.minimum(tid * CPT + g * GRP, N_BLOCKS - GRP)
        return pl.multiple_of(blk * EPB, 8)

    def issue_edges(g, p):
        off = off_of(g)
        pltpu.async_copy(src.at[pl.ds(off, PPG)],
                         srcb.at[pl.ds(p * PPG, PPG)], sem_e)
        pltpu.async_copy(w.at[pl.ds(off, PPG)],
                         wb.at[pl.ds(p * PPG, PPG)], sem_e)
        for b in range(GRP):
            pltpu.async_copy(dst.at[pl.ds(off + b * EPB, EPB)],
                             dstb.at[p * GRP + b], sem_e)

    def drain_edges():
        pltpu.make_async_copy(src.at[pl.ds(0, PPG)],
                              srcb.at[pl.ds(0, PPG)], sem_e).wait()
        pltpu.make_async_copy(w.at[pl.ds(0, PPG)],
                              wb.at[pl.ds(0, PPG)], sem_e).wait()
        for _ in range(GRP):
            pltpu.make_async_copy(dst.at[pl.ds(0, EPB)], dstb.at[0],
                                  sem_e).wait()

    def issue_gathers(p):
        for b in range(GRP):
            o = p * PPG + b * EPB
            pltpu.async_copy(cur.at[cid].at[srcb.at[pl.ds(o, EPB)]],
                             rowsb.at[pl.ds(o, EPB)], sem_g)

    def drain_gathers():
        for _ in range(GRP):
            pltpu.make_async_copy(cur.at[cid, pl.ds(0, EPB)],
                                  rowsb.at[pl.ds(0, EPB)], sem_g).wait()

    def issue_scatters(p):
        for b in range(GRP):
            o = p * PPG + b * EPB
            pltpu.async_copy(rowsb.at[pl.ds(o, EPB)],
                             acc.at[dstb.at[p * GRP + b]], sem_s, add=True)

    def drain_scatters():
        for _ in range(GRP):
            pltpu.make_async_copy(rowsb.at[pl.ds(0, EPB)],
                                  acc.at[pl.ds(0, EPB)], sem_s).wait()

    issue_edges(0, 0)
    drain_edges()
    issue_gathers(0)

    def pair(i, c):
        for p in (0, 1):
            drain_gathers()
            if p == 0:
                @pl.when(i > 0)
                def _():
                    drain_scatters()
            else:
                drain_scatters()
            issue_edges(2 * i + p + 1, p ^ 1)
            for b in range(GRP):
                _mul_block(rowsb, wb, p * PPG + b * EPB)
            issue_scatters(p)
            drain_edges()
            issue_gathers(p ^ 1)
        return c
    lax.fori_loop(0, npair, pair, 0)
    drain_gathers()
    drain_scatters()

    def rem(i, c):
        off = pl.multiple_of((tid * CPT + npair * 2 * GRP + i) * EPB, 8)
        pltpu.sync_copy(src.at[pl.ds(off, EPB)], srcb.at[pl.ds(0, EPB)])
        pltpu.sync_copy(dst.at[pl.ds(off, EPB)], dstb.at[0])
        pltpu.sync_copy(w.at[pl.ds(off, EPB)], wb.at[pl.ds(0, EPB)])
        pltpu.async_copy(cur.at[cid].at[srcb.at[pl.ds(0, EPB)]],
                         rowsb.at[pl.ds(0, EPB)], sem_g).wait()
        _mul_block(rowsb, wb, 0)
        pltpu.async_copy(rowsb.at[pl.ds(0, EPB)], acc.at[dstb.at[0]], sem_s,
                         add=True).wait()
        return c
    lax.fori_loop(0, nch - npair * 2 * GRP, rem, 0)
    plsc.subcore_barrier()
    scope_edge.__exit__(None, None, None)
    scope_dump = jax.named_scope("phase_dump")
    scope_dump.__enter__()

    ndb = jnp.minimum(DBPT, jnp.maximum(0, N_DB - tid * DBPT))

    if not last:
        def dissue(i, c):
            r0 = pl.multiple_of((tid * DBPT + i) * DBLK, 8)
            pltpu.async_copy(acc.at[pl.ds(r0, DBLK)],
                             out.at[cid, pl.ds(r0, DBLK)], sem_s)
            return c
        lax.fori_loop(0, ndb, dissue, 0)

        def ddrain(i, c):
            pltpu.make_async_copy(acc.at[pl.ds(0, DBLK)],
                                  out.at[cid, pl.ds(0, DBLK)], sem_s).wait()
            return c
        lax.fori_loop(0, ndb, ddrain, 0)
    else:
        def dsum(i, c):
            r0 = pl.multiple_of((tid * DBPT + i) * DBLK, 8)
            c1 = pltpu.async_copy(x1.at[cid, pl.ds(r0, DBLK)],
                                  rowsb.at[pl.ds(0, DBLK)], sem_e)
            c2 = pltpu.async_copy(x2.at[cid, pl.ds(r0, DBLK)],
                                  rowsb.at[pl.ds(DBLK, DBLK)], sem_e)
            c3 = pltpu.async_copy(acc.at[pl.ds(r0, DBLK)],
                                  rowsb.at[pl.ds(2 * DBLK, DBLK)], sem_g)
            c1.wait(); c2.wait(); c3.wait()

            def addrow(r, cc):
                rowsb[r, :] = (rowsb[r, :] + rowsb[DBLK + r, :]
                               + rowsb[2 * DBLK + r, :])
                return cc
            lax.fori_loop(0, DBLK, addrow, 0, unroll=8)
            pltpu.async_copy(rowsb.at[pl.ds(0, DBLK)],
                             out.at[cid, pl.ds(r0, DBLK)], sem_s).wait()
            return c
        lax.fori_loop(0, ndb, dsum, 0)
    plsc.subcore_barrier()
    scope_dump.__exit__(None, None, None)


def _sc_body(emb2, src, dst, w, x1s, x2s, sums,
             srcb, dstb, wb, rowsb, acc, sem_e, sem_g, sem_s):
    cid = lax.axis_index("c")
    tid = lax.axis_index("s")
    _layer(cid, tid, emb2, x1s, src, dst, w, False, x1s, x2s,
           srcb, dstb, wb, rowsb, acc, sem_e, sem_g, sem_s)
    _layer(cid, tid, x1s, x2s, src, dst, w, False, x1s, x2s,
           srcb, dstb, wb, rowsb, acc, sem_e, sem_g, sem_s)
    _layer(cid, tid, x2s, sums, src, dst, w, True, x1s, x2s,
           srcb, dstb, wb, rowsb, acc, sem_e, sem_g, sem_s)


_stk = jax.ShapeDtypeStruct((2, N_NODES, HALF_DIM), jnp.float32)

_sc_call = pl.kernel(
    _sc_body,
    out_type=(_stk,) * 3,
    mesh=plsc.VectorSubcoreMesh(core_axis_name="c", subcore_axis_name="s"),
    scratch_types=[
        pltpu.VMEM((2 * PPG,), jnp.int32),
        pltpu.VMEM((2 * GRP, EPB), jnp.int32),
        pltpu.VMEM((2 * PPG,), jnp.float32),
        pltpu.VMEM((2 * PPG, HALF_DIM), jnp.float32),
        pltpu.VMEM_SHARED((N_NODES, HALF_DIM), jnp.float32),
        pltpu.SemaphoreType.DMA,
        pltpu.SemaphoreType.DMA,
        pltpu.SemaphoreType.DMA,
    ],
    compiler_params=pltpu.CompilerParams(use_tc_tiling_on_sc=False),
)


_BLK = 1000


def _mean_body(emb, sums, out):
    out[:, :HALF_DIM] = (emb[:, :HALF_DIM] + sums[0]) * 0.25
    out[:, HALF_DIM:] = (emb[:, HALF_DIM:] + sums[1]) * 0.25


_mean_call = pl.pallas_call(
    _mean_body,
    out_shape=jax.ShapeDtypeStruct((N_NODES, EMBED_DIM), jnp.float32),
    grid=(N_NODES // _BLK,),
    in_specs=[pl.BlockSpec((_BLK, EMBED_DIM), lambda i: (i, 0)),
              pl.BlockSpec((2, _BLK, HALF_DIM), lambda i: (0, i, 0))],
    out_specs=pl.BlockSpec((_BLK, EMBED_DIM), lambda i: (i, 0)),
)


def kernel(embed, edge_index, edge_weight):
    emb2 = jnp.stack([embed[:, :HALF_DIM], embed[:, HALF_DIM:]], axis=0)
    src = edge_index[0]
    dst = edge_index[1]
    _, _, sums = _sc_call(emb2, src, dst, edge_weight)
    return _mean_call(embed, sums)


# speedup vs baseline: 1.0006x; 1.0006x over previous
import functools

import jax
import jax.numpy as jnp
from jax import lax
from jax.experimental import pallas as pl
from jax.experimental.pallas import tpu as pltpu
from jax.experimental.pallas import tpu_sc as plsc

N_NODES = 100000
EMBED_DIM = 32
HALF_DIM = 16
N_EDGES = 1600000

EPB = 128
N_BLOCKS = N_EDGES // EPB
NS = 16
CPT = -(-N_BLOCKS // NS)
GRP = 4
PPG = GRP * EPB
DBLK = 200
N_DB = N_NODES // DBLK
DBPT = -(-N_DB // NS)


def _mul_block(rowsb, wb, wrow, roff):
    def grp16(g, c):
        wv = wb[wrow, pl.ds(g * HALF_DIM, HALF_DIM)]
        for k in range(HALF_DIM):
            e = roff + g * HALF_DIM + k
            splat = jnp.take_along_axis(
                wv, jnp.full((HALF_DIM,), k, jnp.int32), axis=0,
                mode="promise_in_bounds")
            rowsb[e, :] = rowsb[e, :] * splat
        return c
    lax.fori_loop(0, EPB // HALF_DIM, grp16, 0)


def _layer(cid, tid, cur, out, src2, dst2, w2, last, emb2, x1, x2,
           srcb, dstb, wb, rowsb, acc, sem_e, sem_g, sem_s):
    def zb(e, c):
        rowsb[e, :] = jnp.zeros((HALF_DIM,), jnp.float32)
        return c
    lax.fori_loop(0, DBLK, zb, 0, unroll=8)
    nzb = jnp.minimum(DBPT, jnp.maximum(0, N_DB - tid * DBPT))

    def zissue(i, c):
        r0 = pl.multiple_of((tid * DBPT + i) * DBLK, 8)
        pltpu.async_copy(rowsb.at[pl.ds(0, DBLK)], acc.at[pl.ds(r0, DBLK)],
                         sem_s)
        return c
    lax.fori_loop(0, nzb, zissue, 0)

    def zdrain(i, c):
        pltpu.make_async_copy(rowsb.at[pl.ds(0, DBLK)],
                              acc.at[pl.ds(0, DBLK)], sem_s).wait()
        return c
    lax.fori_loop(0, nzb, zdrain, 0)
    plsc.subcore_barrier()

    nch = jnp.minimum(CPT, jnp.maximum(0, N_BLOCKS - tid * CPT))
    ngrp = nch // GRP
    npair = ngrp // 2

    def blk_of(g):
        return jnp.minimum(tid * CPT + g * GRP, N_BLOCKS - GRP)

    def issue_edges(g, p):
        blk = blk_of(g)
        pltpu.async_copy(src2.at[pl.ds(blk, GRP)],
                         srcb.at[pl.ds(p * GRP, GRP)], sem_e)
        pltpu.async_copy(w2.at[pl.ds(blk, GRP)],
                         wb.at[pl.ds(p * GRP, GRP)], sem_e)
        pltpu.async_copy(dst2.at[pl.ds(blk, GRP)],
                         dstb.at[pl.ds(p * GRP, GRP)], sem_e)

    def drain_edges():
        pltpu.make_async_copy(src2.at[pl.ds(0, GRP)],
                              srcb.at[pl.ds(0, GRP)], sem_e).wait()
        pltpu.make_async_copy(w2.at[pl.ds(0, GRP)],
                              wb.at[pl.ds(0, GRP)], sem_e).wait()
        pltpu.make_async_copy(dst2.at[pl.ds(0, GRP)],
                              dstb.at[pl.ds(0, GRP)], sem_e).wait()

    def issue_gathers(p):
        for b in range(GRP):
            pltpu.async_copy(cur.at[cid].at[srcb.at[p * GRP + b]],
                             rowsb.at[pl.ds((p * GRP + b) * EPB, EPB)], sem_g)

    def drain_gathers():
        for _ in range(GRP):
            pltpu.make_async_copy(cur.at[cid, pl.ds(0, EPB)],
                                  rowsb.at[pl.ds(0, EPB)], sem_g).wait()

    def issue_scatters(p):
        for b in range(GRP):
            pltpu.async_copy(rowsb.at[pl.ds((p * GRP + b) * EPB, EPB)],
                             acc.at[dstb.at[p * GRP + b]], sem_s, add=True)

    def drain_scatters():
        for _ in range(GRP):
            pltpu.make_async_copy(rowsb.at[pl.ds(0, EPB)],
                                  acc.at[pl.ds(0, EPB)], sem_s).wait()

    issue_edges(0, 0)
    drain_edges()
    issue_gathers(0)

    def pair(i, c):
        for p in (0, 1):
            drain_gathers()
            if p == 0:
                @pl.when(i > 0)
                def _():
                    drain_scatters()
            else:
                drain_scatters()
            issue_edges(2 * i + p + 1, p ^ 1)
            for b in range(GRP):
                _mul_block(rowsb, wb, p * GRP + b, (p * GRP + b) * EPB)
            issue_scatters(p)
            drain_edges()
            issue_gathers(p ^ 1)
        return c
    lax.fori_loop(0, npair, pair, 0)
    drain_gathers()
    drain_scatters()

    def rem(i, c):
        blk = tid * CPT + npair * 2 * GRP + i
        pltpu.sync_copy(src2.at[pl.ds(blk, 1)], srcb.at[pl.ds(0, 1)])
        pltpu.sync_copy(dst2.at[pl.ds(blk, 1)], dstb.at[pl.ds(0, 1)])
        pltpu.sync_copy(w2.at[pl.ds(blk, 1)], wb.at[pl.ds(0, 1)])
        pltpu.async_copy(cur.at[cid].at[srcb.at[0]],
                         rowsb.at[pl.ds(0, EPB)], sem_g).wait()
        _mul_block(rowsb, wb, 0, 0)
        pltpu.async_copy(rowsb.at[pl.ds(0, EPB)], acc.at[dstb.at[0]], sem_s,
                         add=True).wait()
        return c
    lax.fori_loop(0, nch - npair * 2 * GRP, rem, 0)
    plsc.subcore_barrier()

    ndb = jnp.minimum(DBPT, jnp.maximum(0, N_DB - tid * DBPT))

    if not last:
        def dissue(i, c):
            r0 = pl.multiple_of((tid * DBPT + i) * DBLK, 8)
            pltpu.async_copy(acc.at[pl.ds(r0, DBLK)],
                             out.at[cid, pl.ds(r0, DBLK)], sem_s)
            return c
        lax.fori_loop(0, ndb, dissue, 0)

        def ddrain(i, c):
            pltpu.make_async_copy(acc.at[pl.ds(0, DBLK)],
                                  out.at[cid, pl.ds(0, DBLK)], sem_s).wait()
            return c
        lax.fori_loop(0, ndb, ddrain, 0)
    else:
        def dsum(i, c):
            r0 = pl.multiple_of((tid * DBPT + i) * DBLK, 8)
            c1 = pltpu.async_copy(x1.at[cid, pl.ds(r0, DBLK)],
                                  rowsb.at[pl.ds(0, DBLK)], sem_e)
            c2 = pltpu.async_copy(x2.at[cid, pl.ds(r0, DBLK)],
                                  rowsb.at[pl.ds(DBLK, DBLK)], sem_e)
            c3 = pltpu.async_copy(emb2.at[cid, pl.ds(r0, DBLK)],
                                  rowsb.at[pl.ds(2 * DBLK, DBLK)], sem_e)
            c4 = pltpu.async_copy(acc.at[pl.ds(r0, DBLK)],
                                  rowsb.at[pl.ds(3 * DBLK, DBLK)], sem_g)
            c1.wait(); c2.wait(); c3.wait(); c4.wait()

            def addrow(r, cc):
                rowsb[r, :] = (rowsb[r, :] + rowsb[DBLK + r, :]
                               + rowsb[2 * DBLK + r, :]
                               + rowsb[3 * DBLK + r, :]) * 0.25
                return cc
            lax.fori_loop(0, DBLK, addrow, 0, unroll=8)
            pltpu.async_copy(rowsb.at[pl.ds(0, DBLK)],
                             out.at[cid, pl.ds(r0, DBLK)], sem_s).wait()
            return c
        lax.fori_loop(0, ndb, dsum, 0)
    plsc.subcore_barrier()


def _sc_body(emb2, src2, dst2, w2, x1s, x2s, outs,
             srcb, dstb, wb, rowsb, acc, sem_e, sem_g, sem_s):
    cid = lax.axis_index("c")
    tid = lax.axis_index("s")
    _layer(cid, tid, emb2, x1s, src2, dst2, w2, False, emb2, x1s, x2s,
           srcb, dstb, wb, rowsb, acc, sem_e, sem_g, sem_s)
    _layer(cid, tid, x1s, x2s, src2, dst2, w2, False, emb2, x1s, x2s,
           srcb, dstb, wb, rowsb, acc, sem_e, sem_g, sem_s)
    _layer(cid, tid, x2s, outs, src2, dst2, w2, True, emb2, x1s, x2s,
           srcb, dstb, wb, rowsb, acc, sem_e, sem_g, sem_s)


_stk = jax.ShapeDtypeStruct((2, N_NODES, HALF_DIM), jnp.float32)

_sc_call = pl.kernel(
    _sc_body,
    out_type=(_stk,) * 3,
    mesh=plsc.VectorSubcoreMesh(core_axis_name="c", subcore_axis_name="s"),
    scratch_types=[
        pltpu.VMEM((2 * GRP, EPB), jnp.int32),
        pltpu.VMEM((2 * GRP, EPB), jnp.int32),
        pltpu.VMEM((2 * GRP, EPB), jnp.float32),
        pltpu.VMEM((2 * PPG, HALF_DIM), jnp.float32),
        pltpu.VMEM_SHARED((N_NODES, HALF_DIM), jnp.float32),
        pltpu.SemaphoreType.DMA,
        pltpu.SemaphoreType.DMA,
        pltpu.SemaphoreType.DMA,
    ],
    compiler_params=pltpu.CompilerParams(use_tc_tiling_on_sc=False),
)


def kernel(embed, edge_index, edge_weight):
    emb2 = jnp.stack([embed[:, :HALF_DIM], embed[:, HALF_DIM:]], axis=0)
    src2 = edge_index[0].reshape(N_BLOCKS, EPB)
    dst2 = edge_index[1].reshape(N_BLOCKS, EPB)
    w2 = edge_weight.reshape(N_BLOCKS, EPB)
    _, _, outs = _sc_call(emb2, src2, dst2, w2)
    return jnp.concatenate([outs[0], outs[1]], axis=1)


# speedup vs baseline: 1.1088x; 1.1081x over previous
import functools

import jax
import jax.numpy as jnp
from jax import lax
from jax.experimental import pallas as pl
from jax.experimental.pallas import tpu as pltpu
from jax.experimental.pallas import tpu_sc as plsc

N_NODES = 100000
EMBED_DIM = 32
HALF_DIM = 16
N_EDGES = 1600000

EPB = 128
N_BLOCKS = N_EDGES // EPB
NS = 16
CPT = -(-N_BLOCKS // NS)
GRP = 5
PPG = GRP * EPB
DBLK = 200
N_DB = N_NODES // DBLK
DBPT = -(-N_DB // NS)


def _mul_block(rowsb, wb, wrow, roff):
    def grp16(g, c):
        wv = wb[wrow, pl.ds(g * HALF_DIM, HALF_DIM)]
        for k in range(HALF_DIM):
            e = roff + g * HALF_DIM + k
            splat = jnp.take_along_axis(
                wv, jnp.full((HALF_DIM,), k, jnp.int32), axis=0,
                mode="promise_in_bounds")
            rowsb[e, :] = rowsb[e, :] * splat
        return c
    lax.fori_loop(0, EPB // HALF_DIM, grp16, 0)


def _layer(cid, tid, cur, out, src2, dst2, w2, last, emb2, x1, x2,
           srcb, dstb, wb, rowsb, acc, sem_e, sem_g, sem_s):
    def zb(e, c):
        rowsb[e, :] = jnp.zeros((HALF_DIM,), jnp.float32)
        return c
    lax.fori_loop(0, DBLK, zb, 0, unroll=8)
    nzb = jnp.minimum(DBPT, jnp.maximum(0, N_DB - tid * DBPT))

    def zissue(i, c):
        r0 = pl.multiple_of((tid * DBPT + i) * DBLK, 8)
        pltpu.async_copy(rowsb.at[pl.ds(0, DBLK)], acc.at[pl.ds(r0, DBLK)],
                         sem_s)
        return c
    lax.fori_loop(0, nzb, zissue, 0)

    def zdrain(i, c):
        pltpu.make_async_copy(rowsb.at[pl.ds(0, DBLK)],
                              acc.at[pl.ds(0, DBLK)], sem_s).wait()
        return c
    lax.fori_loop(0, nzb, zdrain, 0)
    plsc.subcore_barrier()

    nch = jnp.minimum(CPT, jnp.maximum(0, N_BLOCKS - tid * CPT))
    ngrp = nch // GRP
    npair = ngrp // 2

    def blk_of(g):
        return jnp.minimum(tid * CPT + g * GRP, N_BLOCKS - GRP)

    def issue_edges(g, p):
        blk = blk_of(g)
        pltpu.async_copy(src2.at[pl.ds(blk, GRP)],
                         srcb.at[pl.ds(p * GRP, GRP)], sem_e)
        pltpu.async_copy(w2.at[pl.ds(blk, GRP)],
                         wb.at[pl.ds(p * GRP, GRP)], sem_e)
        pltpu.async_copy(dst2.at[pl.ds(blk, GRP)],
                         dstb.at[pl.ds(p * GRP, GRP)], sem_e)

    def drain_edges():
        pltpu.make_async_copy(src2.at[pl.ds(0, GRP)],
                              srcb.at[pl.ds(0, GRP)], sem_e).wait()
        pltpu.make_async_copy(w2.at[pl.ds(0, GRP)],
                              wb.at[pl.ds(0, GRP)], sem_e).wait()
        pltpu.make_async_copy(dst2.at[pl.ds(0, GRP)],
                              dstb.at[pl.ds(0, GRP)], sem_e).wait()

    def issue_gathers(p):
        for b in range(GRP):
            pltpu.async_copy(cur.at[cid].at[srcb.at[p * GRP + b]],
                             rowsb.at[pl.ds((p * GRP + b) * EPB, EPB)], sem_g)

    def drain_gathers():
        for _ in range(GRP):
            pltpu.make_async_copy(cur.at[cid, pl.ds(0, EPB)],
                                  rowsb.at[pl.ds(0, EPB)], sem_g).wait()

    def issue_scatters(p):
        for b in range(GRP):
            pltpu.async_copy(rowsb.at[pl.ds((p * GRP + b) * EPB, EPB)],
                             acc.at[dstb.at[p * GRP + b]], sem_s, add=True)

    def drain_scatters():
        for _ in range(GRP):
            pltpu.make_async_copy(rowsb.at[pl.ds(0, EPB)],
                                  acc.at[pl.ds(0, EPB)], sem_s).wait()

    issue_edges(0, 0)
    drain_edges()
    issue_gathers(0)

    def pair(i, c):
        for p in (0, 1):
            drain_gathers()
            if p == 0:
                @pl.when(i > 0)
                def _():
                    drain_scatters()
            else:
                drain_scatters()
            issue_edges(2 * i + p + 1, p ^ 1)
            for b in range(GRP):
                _mul_block(rowsb, wb, p * GRP + b, (p * GRP + b) * EPB)
            issue_scatters(p)
            drain_edges()
            issue_gathers(p ^ 1)
        return c
    lax.fori_loop(0, npair, pair, 0)
    drain_gathers()
    drain_scatters()

    def rem(i, c):
        blk = tid * CPT + npair * 2 * GRP + i
        pltpu.sync_copy(src2.at[pl.ds(blk, 1)], srcb.at[pl.ds(0, 1)])
        pltpu.sync_copy(dst2.at[pl.ds(blk, 1)], dstb.at[pl.ds(0, 1)])
        pltpu.sync_copy(w2.at[pl.ds(blk, 1)], wb.at[pl.ds(0, 1)])
        pltpu.async_copy(cur.at[cid].at[srcb.at[0]],
                         rowsb.at[pl.ds(0, EPB)], sem_g).wait()
        _mul_block(rowsb, wb, 0, 0)
        pltpu.async_copy(rowsb.at[pl.ds(0, EPB)], acc.at[dstb.at[0]], sem_s,
                         add=True).wait()
        return c
    lax.fori_loop(0, nch - npair * 2 * GRP, rem, 0)
    plsc.subcore_barrier()

    ndb = jnp.minimum(DBPT, jnp.maximum(0, N_DB - tid * DBPT))

    if not last:
        def dissue(i, c):
            r0 = pl.multiple_of((tid * DBPT + i) * DBLK, 8)
            pltpu.async_copy(acc.at[pl.ds(r0, DBLK)],
                             out.at[cid, pl.ds(r0, DBLK)], sem_s)
            return c
        lax.fori_loop(0, ndb, dissue, 0)

        def ddrain(i, c):
            pltpu.make_async_copy(acc.at[pl.ds(0, DBLK)],
                                  out.at[cid, pl.ds(0, DBLK)], sem_s).wait()
            return c
        lax.fori_loop(0, ndb, ddrain, 0)
    else:
        def dsum(i, c):
            r0 = pl.multiple_of((tid * DBPT + i) * DBLK, 8)
            c1 = pltpu.async_copy(x1.at[cid, pl.ds(r0, DBLK)],
                                  rowsb.at[pl.ds(0, DBLK)], sem_e)
            c2 = pltpu.async_copy(x2.at[cid, pl.ds(r0, DBLK)],
                                  rowsb.at[pl.ds(DBLK, DBLK)], sem_e)
            c3 = pltpu.async_copy(emb2.at[cid, pl.ds(r0, DBLK)],
                                  rowsb.at[pl.ds(2 * DBLK, DBLK)], sem_e)
            c4 = pltpu.async_copy(acc.at[pl.ds(r0, DBLK)],
                                  rowsb.at[pl.ds(3 * DBLK, DBLK)], sem_g)
            c1.wait(); c2.wait(); c3.wait(); c4.wait()

            def addrow(r, cc):
                rowsb[r, :] = (rowsb[r, :] + rowsb[DBLK + r, :]
                               + rowsb[2 * DBLK + r, :]
                               + rowsb[3 * DBLK + r, :]) * 0.25
                return cc
            lax.fori_loop(0, DBLK, addrow, 0, unroll=8)
            pltpu.async_copy(rowsb.at[pl.ds(0, DBLK)],
                             out.at[cid, pl.ds(r0, DBLK)], sem_s).wait()
            return c
        lax.fori_loop(0, ndb, dsum, 0)
    plsc.subcore_barrier()


def _sc_body(emb2, src2, dst2, w2, x1s, x2s, outs,
             srcb, dstb, wb, rowsb, acc, sem_e, sem_g, sem_s):
    cid = lax.axis_index("c")
    tid = lax.axis_index("s")
    _layer(cid, tid, emb2, x1s, src2, dst2, w2, False, emb2, x1s, x2s,
           srcb, dstb, wb, rowsb, acc, sem_e, sem_g, sem_s)
    _layer(cid, tid, x1s, x2s, src2, dst2, w2, False, emb2, x1s, x2s,
           srcb, dstb, wb, rowsb, acc, sem_e, sem_g, sem_s)
    _layer(cid, tid, x2s, outs, src2, dst2, w2, True, emb2, x1s, x2s,
           srcb, dstb, wb, rowsb, acc, sem_e, sem_g, sem_s)


_stk = jax.ShapeDtypeStruct((2, N_NODES, HALF_DIM), jnp.float32)

_sc_call = pl.kernel(
    _sc_body,
    out_type=(_stk,) * 3,
    mesh=plsc.VectorSubcoreMesh(core_axis_name="c", subcore_axis_name="s"),
    scratch_types=[
        pltpu.VMEM((2 * GRP, EPB), jnp.int32),
        pltpu.VMEM((2 * GRP, EPB), jnp.int32),
        pltpu.VMEM((2 * GRP, EPB), jnp.float32),
        pltpu.VMEM((2 * PPG, HALF_DIM), jnp.float32),
        pltpu.VMEM_SHARED((N_NODES, HALF_DIM), jnp.float32),
        pltpu.SemaphoreType.DMA,
        pltpu.SemaphoreType.DMA,
        pltpu.SemaphoreType.DMA,
    ],
    compiler_params=pltpu.CompilerParams(use_tc_tiling_on_sc=False),
)


def kernel(embed, edge_index, edge_weight):
    emb2 = jnp.stack([embed[:, :HALF_DIM], embed[:, HALF_DIM:]], axis=0)
    src2 = edge_index[0].reshape(N_BLOCKS, EPB)
    dst2 = edge_index[1].reshape(N_BLOCKS, EPB)
    w2 = edge_weight.reshape(N_BLOCKS, EPB)
    _, _, outs = _sc_call(emb2, src2, dst2, w2)
    return jnp.concatenate([outs[0], outs[1]], axis=1)


# speedup vs baseline: 1.1655x; 1.0512x over previous
import functools

import jax
import jax.numpy as jnp
from jax import lax
from jax.experimental import pallas as pl
from jax.experimental.pallas import tpu as pltpu
from jax.experimental.pallas import tpu_sc as plsc

N_NODES = 100000
EMBED_DIM = 32
HALF_DIM = 16
N_EDGES = 1600000

EPB = 128
N_BLOCKS = N_EDGES // EPB
NS = 16
CPT = -(-N_BLOCKS // NS)
GRP = 6
PPG = GRP * EPB
DBLK = 200
N_DB = N_NODES // DBLK
DBPT = -(-N_DB // NS)


def _mul_block(rowsb, wb, wrow, roff):
    def grp16(g, c):
        wv = wb[wrow, pl.ds(g * HALF_DIM, HALF_DIM)]
        for k in range(HALF_DIM):
            e = roff + g * HALF_DIM + k
            splat = jnp.take_along_axis(
                wv, jnp.full((HALF_DIM,), k, jnp.int32), axis=0,
                mode="promise_in_bounds")
            rowsb[e, :] = rowsb[e, :] * splat
        return c
    lax.fori_loop(0, EPB // HALF_DIM, grp16, 0)


def _layer(cid, tid, cur, out, src2, dst2, w2, last, emb2, x1, x2,
           srcb, dstb, wb, rowsb, acc, sem_e, sem_g, sem_s):
    def zb(e, c):
        rowsb[e, :] = jnp.zeros((HALF_DIM,), jnp.float32)
        return c
    lax.fori_loop(0, DBLK, zb, 0, unroll=8)
    nzb = jnp.minimum(DBPT, jnp.maximum(0, N_DB - tid * DBPT))

    def zissue(i, c):
        r0 = pl.multiple_of((tid * DBPT + i) * DBLK, 8)
        pltpu.async_copy(rowsb.at[pl.ds(0, DBLK)], acc.at[pl.ds(r0, DBLK)],
                         sem_s)
        return c
    lax.fori_loop(0, nzb, zissue, 0)

    def zdrain(i, c):
        pltpu.make_async_copy(rowsb.at[pl.ds(0, DBLK)],
                              acc.at[pl.ds(0, DBLK)], sem_s).wait()
        return c
    lax.fori_loop(0, nzb, zdrain, 0)
    plsc.subcore_barrier()

    nch = jnp.minimum(CPT, jnp.maximum(0, N_BLOCKS - tid * CPT))
    ngrp = nch // GRP
    npair = ngrp // 2

    def blk_of(g):
        return jnp.minimum(tid * CPT + g * GRP, N_BLOCKS - GRP)

    def issue_edges(g, p):
        blk = blk_of(g)
        pltpu.async_copy(src2.at[pl.ds(blk, GRP)],
                         srcb.at[pl.ds(p * GRP, GRP)], sem_e)
        pltpu.async_copy(w2.at[pl.ds(blk, GRP)],
                         wb.at[pl.ds(p * GRP, GRP)], sem_e)
        pltpu.async_copy(dst2.at[pl.ds(blk, GRP)],
                         dstb.at[pl.ds(p * GRP, GRP)], sem_e)

    def drain_edges():
        pltpu.make_async_copy(src2.at[pl.ds(0, GRP)],
                              srcb.at[pl.ds(0, GRP)], sem_e).wait()
        pltpu.make_async_copy(w2.at[pl.ds(0, GRP)],
                              wb.at[pl.ds(0, GRP)], sem_e).wait()
        pltpu.make_async_copy(dst2.at[pl.ds(0, GRP)],
                              dstb.at[pl.ds(0, GRP)], sem_e).wait()

    def issue_gathers(p):
        for b in range(GRP):
            pltpu.async_copy(cur.at[cid].at[srcb.at[p * GRP + b]],
                             rowsb.at[pl.ds((p * GRP + b) * EPB, EPB)], sem_g)

    def drain_gathers():
        for _ in range(GRP):
            pltpu.make_async_copy(cur.at[cid, pl.ds(0, EPB)],
                                  rowsb.at[pl.ds(0, EPB)], sem_g).wait()

    def issue_scatters(p):
        for b in range(GRP):
            pltpu.async_copy(rowsb.at[pl.ds((p * GRP + b) * EPB, EPB)],
                             acc.at[dstb.at[p * GRP + b]], sem_s, add=True)

    def drain_scatters():
        for _ in range(GRP):
            pltpu.make_async_copy(rowsb.at[pl.ds(0, EPB)],
                                  acc.at[pl.ds(0, EPB)], sem_s).wait()

    issue_edges(0, 0)
    drain_edges()
    issue_gathers(0)

    def pair(i, c):
        for p in (0, 1):
            drain_gathers()
            if p == 0:
                @pl.when(i > 0)
                def _():
                    drain_scatters()
            else:
                drain_scatters()
            issue_edges(2 * i + p + 1, p ^ 1)
            for b in range(GRP):
                _mul_block(rowsb, wb, p * GRP + b, (p * GRP + b) * EPB)
            issue_scatters(p)
            drain_edges()
            issue_gathers(p ^ 1)
        return c
    lax.fori_loop(0, npair, pair, 0)
    drain_gathers()
    drain_scatters()

    def rem(i, c):
        blk = tid * CPT + npair * 2 * GRP + i
        pltpu.sync_copy(src2.at[pl.ds(blk, 1)], srcb.at[pl.ds(0, 1)])
        pltpu.sync_copy(dst2.at[pl.ds(blk, 1)], dstb.at[pl.ds(0, 1)])
        pltpu.sync_copy(w2.at[pl.ds(blk, 1)], wb.at[pl.ds(0, 1)])
        pltpu.async_copy(cur.at[cid].at[srcb.at[0]],
                         rowsb.at[pl.ds(0, EPB)], sem_g).wait()
        _mul_block(rowsb, wb, 0, 0)
        pltpu.async_copy(rowsb.at[pl.ds(0, EPB)], acc.at[dstb.at[0]], sem_s,
                         add=True).wait()
        return c
    lax.fori_loop(0, nch - npair * 2 * GRP, rem, 0)
    plsc.subcore_barrier()

    ndb = jnp.minimum(DBPT, jnp.maximum(0, N_DB - tid * DBPT))

    if not last:
        def dissue(i, c):
            r0 = pl.multiple_of((tid * DBPT + i) * DBLK, 8)
            pltpu.async_copy(acc.at[pl.ds(r0, DBLK)],
                             out.at[cid, pl.ds(r0, DBLK)], sem_s)
            return c
        lax.fori_loop(0, ndb, dissue, 0)

        def ddrain(i, c):
            pltpu.make_async_copy(acc.at[pl.ds(0, DBLK)],
                                  out.at[cid, pl.ds(0, DBLK)], sem_s).wait()
            return c
        lax.fori_loop(0, ndb, ddrain, 0)
    else:
        def dsum(i, c):
            r0 = pl.multiple_of((tid * DBPT + i) * DBLK, 8)
            c1 = pltpu.async_copy(x1.at[cid, pl.ds(r0, DBLK)],
                                  rowsb.at[pl.ds(0, DBLK)], sem_e)
            c2 = pltpu.async_copy(x2.at[cid, pl.ds(r0, DBLK)],
                                  rowsb.at[pl.ds(DBLK, DBLK)], sem_e)
            c3 = pltpu.async_copy(emb2.at[cid, pl.ds(r0, DBLK)],
                                  rowsb.at[pl.ds(2 * DBLK, DBLK)], sem_e)
            c4 = pltpu.async_copy(acc.at[pl.ds(r0, DBLK)],
                                  rowsb.at[pl.ds(3 * DBLK, DBLK)], sem_g)
            c1.wait(); c2.wait(); c3.wait(); c4.wait()

            def addrow(r, cc):
                rowsb[r, :] = (rowsb[r, :] + rowsb[DBLK + r, :]
                               + rowsb[2 * DBLK + r, :]
                               + rowsb[3 * DBLK + r, :]) * 0.25
                return cc
            lax.fori_loop(0, DBLK, addrow, 0, unroll=8)
            pltpu.async_copy(rowsb.at[pl.ds(0, DBLK)],
                             out.at[cid, pl.ds(r0, DBLK)], sem_s).wait()
            return c
        lax.fori_loop(0, ndb, dsum, 0)
    plsc.subcore_barrier()


def _sc_body(emb2, src2, dst2, w2, x1s, x2s, outs,
             srcb, dstb, wb, rowsb, acc, sem_e, sem_g, sem_s):
    cid = lax.axis_index("c")
    tid = lax.axis_index("s")
    _layer(cid, tid, emb2, x1s, src2, dst2, w2, False, emb2, x1s, x2s,
           srcb, dstb, wb, rowsb, acc, sem_e, sem_g, sem_s)
    _layer(cid, tid, x1s, x2s, src2, dst2, w2, False, emb2, x1s, x2s,
           srcb, dstb, wb, rowsb, acc, sem_e, sem_g, sem_s)
    _layer(cid, tid, x2s, outs, src2, dst2, w2, True, emb2, x1s, x2s,
           srcb, dstb, wb, rowsb, acc, sem_e, sem_g, sem_s)


_stk = jax.ShapeDtypeStruct((2, N_NODES, HALF_DIM), jnp.float32)

_sc_call = pl.kernel(
    _sc_body,
    out_type=(_stk,) * 3,
    mesh=plsc.VectorSubcoreMesh(core_axis_name="c", subcore_axis_name="s"),
    scratch_types=[
        pltpu.VMEM((2 * GRP, EPB), jnp.int32),
        pltpu.VMEM((2 * GRP, EPB), jnp.int32),
        pltpu.VMEM((2 * GRP, EPB), jnp.float32),
        pltpu.VMEM((2 * PPG, HALF_DIM), jnp.float32),
        pltpu.VMEM_SHARED((N_NODES, HALF_DIM), jnp.float32),
        pltpu.SemaphoreType.DMA,
        pltpu.SemaphoreType.DMA,
        pltpu.SemaphoreType.DMA,
    ],
    compiler_params=pltpu.CompilerParams(use_tc_tiling_on_sc=False),
)


def kernel(embed, edge_index, edge_weight):
    emb2 = jnp.stack([embed[:, :HALF_DIM], embed[:, HALF_DIM:]], axis=0)
    src2 = edge_index[0].reshape(N_BLOCKS, EPB)
    dst2 = edge_index[1].reshape(N_BLOCKS, EPB)
    w2 = edge_weight.reshape(N_BLOCKS, EPB)
    _, _, outs = _sc_call(emb2, src2, dst2, w2)
    return jnp.concatenate([outs[0], outs[1]], axis=1)


# speedup vs baseline: 1.3572x; 1.1645x over previous
import functools

import jax
import jax.numpy as jnp
from jax import lax
from jax.experimental import pallas as pl
from jax.experimental.pallas import tpu as pltpu
from jax.experimental.pallas import tpu_sc as plsc

N_NODES = 100000
EMBED_DIM = 32
HALF_DIM = 16
N_EDGES = 1600000

EPB = 128
N_BLOCKS = N_EDGES // EPB
NS = 16
CPT = -(-N_BLOCKS // NS)
GRP = 6
PPG = GRP * EPB
DBLK = 200
N_DB = N_NODES // DBLK
DBPT = -(-N_DB // NS)


def _mul_block(rowsb, wb, wrow, roff):
    def grp16(g, c):
        wv = wb[wrow, pl.ds(g * HALF_DIM, HALF_DIM)]
        for k in range(HALF_DIM):
            e = roff + g * HALF_DIM + k
            splat = jnp.take_along_axis(
                wv, jnp.full((HALF_DIM,), k, jnp.int32), axis=0,
                mode="promise_in_bounds")
            rowsb[e, :] = rowsb[e, :] * splat
        return c
    lax.fori_loop(0, EPB // HALF_DIM, grp16, 0)


def _layer(cid, tid, cur, out, src2, dst2, w2, last, emb2, x1, x2,
           srcb, dstb, wb, rowsb, acc, sem_src, sem_wd, sem_g, sem_s):
    def zb(e, c):
        rowsb[e, :] = jnp.zeros((HALF_DIM,), jnp.float32)
        return c
    lax.fori_loop(0, DBLK, zb, 0, unroll=8)
    nzb = jnp.minimum(DBPT, jnp.maximum(0, N_DB - tid * DBPT))

    def zissue(i, c):
        r0 = pl.multiple_of((tid * DBPT + i) * DBLK, 8)
        pltpu.async_copy(rowsb.at[pl.ds(0, DBLK)], acc.at[pl.ds(r0, DBLK)],
                         sem_s)
        return c
    lax.fori_loop(0, nzb, zissue, 0)

    def zdrain(i, c):
        pltpu.make_async_copy(rowsb.at[pl.ds(0, DBLK)],
                              acc.at[pl.ds(0, DBLK)], sem_s).wait()
        return c
    lax.fori_loop(0, nzb, zdrain, 0)
    plsc.subcore_barrier()

    nch = jnp.minimum(CPT, jnp.maximum(0, N_BLOCKS - tid * CPT))
    ngrp = nch // GRP
    nquad = ngrp // 4

    def blk_of(g):
        return jnp.minimum(tid * CPT + g * GRP, N_BLOCKS - GRP)

    def issue_src(g, s):
        pltpu.async_copy(src2.at[pl.ds(blk_of(g), GRP)],
                         srcb.at[pl.ds(s * GRP, GRP)], sem_src[s])

    def drain_src(s):
        pltpu.make_async_copy(src2.at[pl.ds(0, GRP)],
                              srcb.at[pl.ds(0, GRP)], sem_src[s]).wait()

    def issue_wd(g, p):
        blk = blk_of(g)
        pltpu.async_copy(w2.at[pl.ds(blk, GRP)],
                         wb.at[pl.ds(p * GRP, GRP)], sem_wd[p])
        pltpu.async_copy(dst2.at[pl.ds(blk, GRP)],
                         dstb.at[pl.ds(p * GRP, GRP)], sem_wd[p])

    def drain_wd(p):
        pltpu.make_async_copy(w2.at[pl.ds(0, GRP)],
                              wb.at[pl.ds(0, GRP)], sem_wd[p]).wait()
        pltpu.make_async_copy(dst2.at[pl.ds(0, GRP)],
                              dstb.at[pl.ds(0, GRP)], sem_wd[p]).wait()

    def issue_gathers(s, p):
        for b in range(GRP):
            pltpu.async_copy(cur.at[cid].at[srcb.at[s * GRP + b]],
                             rowsb.at[pl.ds((p * GRP + b) * EPB, EPB)], sem_g)

    def drain_gathers():
        for _ in range(GRP):
            pltpu.make_async_copy(cur.at[cid, pl.ds(0, EPB)],
                                  rowsb.at[pl.ds(0, EPB)], sem_g).wait()

    def issue_scatters(p):
        for b in range(GRP):
            pltpu.async_copy(rowsb.at[pl.ds((p * GRP + b) * EPB, EPB)],
                             acc.at[dstb.at[p * GRP + b]], sem_s, add=True)

    def drain_scatters():
        for _ in range(GRP):
            pltpu.make_async_copy(rowsb.at[pl.ds(0, EPB)],
                                  acc.at[pl.ds(0, EPB)], sem_s).wait()

    issue_src(0, 0)
    issue_src(1, 1)
    issue_wd(0, 0)
    drain_src(0)
    issue_gathers(0, 0)

    def quad(q, c):
        for j in range(4):
            p = j & 1
            drain_gathers()
            if j == 0:
                @pl.when(q > 0)
                def _():
                    drain_scatters()
            else:
                drain_scatters()
            g = 4 * q + j
            issue_src(g + 2, (j + 2) % 4)
            issue_wd(g + 1, p ^ 1)
            drain_src((j + 1) % 4)
            issue_gathers((j + 1) % 4, p ^ 1)
            drain_wd(p)
            for b in range(GRP):
                _mul_block(rowsb, wb, p * GRP + b, (p * GRP + b) * EPB)
            issue_scatters(p)
        return c
    lax.fori_loop(0, nquad, quad, 0)
    drain_gathers()
    drain_scatters()
    drain_src(1)
    drain_wd(0)

    def rem(i, c):
        blk = tid * CPT + nquad * 4 * GRP + i
        pltpu.sync_copy(src2.at[pl.ds(blk, 1)], srcb.at[pl.ds(0, 1)])
        pltpu.sync_copy(dst2.at[pl.ds(blk, 1)], dstb.at[pl.ds(0, 1)])
        pltpu.sync_copy(w2.at[pl.ds(blk, 1)], wb.at[pl.ds(0, 1)])
        pltpu.async_copy(cur.at[cid].at[srcb.at[0]],
                         rowsb.at[pl.ds(0, EPB)], sem_g).wait()
        _mul_block(rowsb, wb, 0, 0)
        pltpu.async_copy(rowsb.at[pl.ds(0, EPB)], acc.at[dstb.at[0]], sem_s,
                         add=True).wait()
        return c
    lax.fori_loop(0, nch - nquad * 4 * GRP, rem, 0)
    plsc.subcore_barrier()

    ndb = jnp.minimum(DBPT, jnp.maximum(0, N_DB - tid * DBPT))

    if not last:
        def dissue(i, c):
            r0 = pl.multiple_of((tid * DBPT + i) * DBLK, 8)
            pltpu.async_copy(acc.at[pl.ds(r0, DBLK)],
                             out.at[cid, pl.ds(r0, DBLK)], sem_s)
            return c
        lax.fori_loop(0, ndb, dissue, 0)

        def ddrain(i, c):
            pltpu.make_async_copy(acc.at[pl.ds(0, DBLK)],
                                  out.at[cid, pl.ds(0, DBLK)], sem_s).wait()
            return c
        lax.fori_loop(0, ndb, ddrain, 0)
    else:
        def dsum(i, c):
            r0 = pl.multiple_of((tid * DBPT + i) * DBLK, 8)
            c1 = pltpu.async_copy(x1.at[cid, pl.ds(r0, DBLK)],
                                  rowsb.at[pl.ds(0, DBLK)], sem_wd[0])
            c2 = pltpu.async_copy(x2.at[cid, pl.ds(r0, DBLK)],
                                  rowsb.at[pl.ds(DBLK, DBLK)], sem_wd[1])
            c3 = pltpu.async_copy(emb2.at[cid, pl.ds(r0, DBLK)],
                                  rowsb.at[pl.ds(2 * DBLK, DBLK)], sem_src[0])
            c4 = pltpu.async_copy(acc.at[pl.ds(r0, DBLK)],
                                  rowsb.at[pl.ds(3 * DBLK, DBLK)], sem_g)
            c1.wait(); c2.wait(); c3.wait(); c4.wait()

            def addrow(r, cc):
                rowsb[r, :] = (rowsb[r, :] + rowsb[DBLK + r, :]
                               + rowsb[2 * DBLK + r, :]
                               + rowsb[3 * DBLK + r, :]) * 0.25
                return cc
            lax.fori_loop(0, DBLK, addrow, 0, unroll=8)
            pltpu.async_copy(rowsb.at[pl.ds(0, DBLK)],
                             out.at[cid, pl.ds(r0, DBLK)], sem_s).wait()
            return c
        lax.fori_loop(0, ndb, dsum, 0)
    plsc.subcore_barrier()


def _sc_body(emb2, src2, dst2, w2, x1s, x2s, outs,
             srcb, dstb, wb, rowsb, acc, sem_src, sem_wd, sem_g, sem_s):
    cid = lax.axis_index("c")
    tid = lax.axis_index("s")
    _layer(cid, tid, emb2, x1s, src2, dst2, w2, False, emb2, x1s, x2s,
           srcb, dstb, wb, rowsb, acc, sem_src, sem_wd, sem_g, sem_s)
    _layer(cid, tid, x1s, x2s, src2, dst2, w2, False, emb2, x1s, x2s,
           srcb, dstb, wb, rowsb, acc, sem_src, sem_wd, sem_g, sem_s)
    _layer(cid, tid, x2s, outs, src2, dst2, w2, True, emb2, x1s, x2s,
           srcb, dstb, wb, rowsb, acc, sem_src, sem_wd, sem_g, sem_s)


_stk = jax.ShapeDtypeStruct((2, N_NODES, HALF_DIM), jnp.float32)

_sc_call = pl.kernel(
    _sc_body,
    out_type=(_stk,) * 3,
    mesh=plsc.VectorSubcoreMesh(core_axis_name="c", subcore_axis_name="s"),
    scratch_types=[
        pltpu.VMEM((4 * GRP, EPB), jnp.int32),
        pltpu.VMEM((2 * GRP, EPB), jnp.int32),
        pltpu.VMEM((2 * GRP, EPB), jnp.float32),
        pltpu.VMEM((2 * PPG, HALF_DIM), jnp.float32),
        pltpu.VMEM_SHARED((N_NODES, HALF_DIM), jnp.float32),
        [pltpu.SemaphoreType.DMA] * 4,
        [pltpu.SemaphoreType.DMA] * 2,
        pltpu.SemaphoreType.DMA,
        pltpu.SemaphoreType.DMA,
    ],
    compiler_params=pltpu.CompilerParams(use_tc_tiling_on_sc=False),
)


def kernel(embed, edge_index, edge_weight):
    emb2 = jnp.stack([embed[:, :HALF_DIM], embed[:, HALF_DIM:]], axis=0)
    src2 = edge_index[0].reshape(N_BLOCKS, EPB)
    dst2 = edge_index[1].reshape(N_BLOCKS, EPB)
    w2 = edge_weight.reshape(N_BLOCKS, EPB)
    _, _, outs = _sc_call(emb2, src2, dst2, w2)
    return jnp.concatenate([outs[0], outs[1]], axis=1)
